# baseline (speedup 1.0000x reference)
import functools

import jax
import jax.numpy as jnp
from jax import lax
from jax.experimental import pallas as pl
from jax.experimental.pallas import tpu as pltpu
from jax.experimental.pallas import tpu_sc as plsc

_N = 4096
_R = 64
_E = 64
_V = 100000
_L = 16
_NT = 782
_TPW = 25
_RT = 4
_NR = 7
_RC = _RT * 128
_JUNK = _N


@jax.jit
def _embed_concat(x, info, table):
    sc = plsc.get_sparse_core_info()
    nc, ns = sc.num_cores, sc.num_subcores
    nw = nc * ns

    tableT = table.T

    mesh = plsc.VectorSubcoreMesh(core_axis_name="c", subcore_axis_name="s")

    @functools.partial(
        pl.kernel,
        mesh=mesh,
        out_type=jax.ShapeDtypeStruct((_N + _L, _E + _R), jnp.float32),
        scratch_types=[
            pltpu.VMEM((_N,), jnp.int32),
            pltpu.VMEM((_N + _L,), jnp.int32),
            pltpu.VMEM((_N + _L,), jnp.int32),
            pltpu.VMEM((_N + _L,), jnp.int32),
            pltpu.VMEM((_N + _L,), jnp.int32),
            pltpu.VMEM((2, _E, _RC), jnp.float32),
            pltpu.VMEM((_L * 129 + _L,), jnp.float32),
            pltpu.VMEM((128, _E + _R), jnp.float32),
            pltpu.VMEM((8, _L), jnp.int32),
            pltpu.SemaphoreType.DMA,
            pltpu.SemaphoreType.DMA,
            pltpu.SemaphoreType.DMA,
        ],
        compiler_params=pltpu.CompilerParams(needs_layout_passes=False),
    )
    def k(idx_hbm, tableT_hbm, out_hbm,
          idx_v, hcol_v, hsmp_v, rcol_v, rsmp_v, stage_v, pitch_v, f_v,
          fsmp_v, sem_a, sem_b, sem_s):
        wid = lax.axis_index("s") * nc + lax.axis_index("c")
        tlo = _TPW * wid
        wtiles = jnp.minimum(_TPW, _NT - tlo)
        clo = tlo * 128
        wlen = wtiles * 128

        iota = lax.iota(jnp.int32, _L)
        junk_vec = jnp.full((_L,), _JUNK, jnp.int32)
        zero_vec = jnp.zeros((_L,), jnp.int32)

        pltpu.sync_copy(idx_hbm, idx_v)

        clo_vec = jnp.full((_L,), clo, jnp.int32)
        chi_vec = jnp.full((_L,), clo + wlen, jnp.int32)

        def scan_body(j, off):
            v = idx_v[pl.ds(j * _L, _L)]
            m = (v >= clo_vec) & (v < chi_vec)
            n = jnp.max(plsc.all_reduce_population_count(m))
            plsc.store_compressed(hcol_v.at[pl.ds(off, _L)], v - clo_vec,
                                  mask=m)
            plsc.store_compressed(hsmp_v.at[pl.ds(off, _L)], j * _L + iota,
                                  mask=m)
            return off + n

        cnt = lax.fori_loop(0, _N // _L, scan_body, 0)
        hsmp_v[pl.ds(cnt, _L)] = junk_vec
        hcol_v[pl.ds(cnt, _L)] = zero_vec

        def reset_fsmp():
            for h in range(8):
                plsc.store_scatter(
                    fsmp_v, [jnp.full((_L,), h, jnp.int32), iota], junk_vec)

        reset_fsmp()

        def do_flush():
            copies = []
            for h in range(8):
                copies.append(pltpu.async_copy(
                    f_v.at[pl.ds(_L * h, _L)],
                    out_hbm.at[fsmp_v.at[h]],
                    sem_s,
                ))
            for c in copies:
                c.wait()
            reset_fsmp()

        sems = (sem_a, sem_b)

        def round_dma(r):
            c0loc = jnp.maximum(0, jnp.minimum(_RC * r, wlen - _RC))
            gcol = pl.multiple_of(clo + c0loc, 128)
            return c0loc, pltpu.async_copy(
                tableT_hbm.at[:, pl.ds(gcol, _RC)],
                stage_v.at[r % 2],
                sems[r % 2],
            )

        sbase0, dma0 = round_dma(0)
        pending = [(sbase0, dma0)]

        col_consts = [jnp.full((_L,), 16 * q, jnp.int32) + iota
                      for q in range(8)]

        cf = 0
        for r in range(_NR):
            sbase, dma = pending.pop()
            if r + 1 < _NR:
                pending.append(round_dma(r + 1))

            rlo_vec = jnp.full((_L,), _RC * r, jnp.int32)
            rhi = jnp.minimum(_RC * (r + 1), wlen)
            rhi_vec = jnp.full((_L,), rhi, jnp.int32)
            sbase_vec = jnp.full((_L,), sbase, jnp.int32)

            def rscan_body(j, roff):
                hc = hcol_v[pl.ds(j * _L, _L)]
                hs = hsmp_v[pl.ds(j * _L, _L)]
                m = (hc >= rlo_vec) & (hc < rhi_vec)
                n = jnp.max(plsc.all_reduce_population_count(m))
                plsc.store_compressed(rcol_v.at[pl.ds(roff, _L)],
                                      hc - sbase_vec, mask=m)
                plsc.store_compressed(rsmp_v.at[pl.ds(roff, _L)], hs, mask=m)
                return roff + n

            rcnt = lax.fori_loop(0, (cnt + _L - 1) // _L, rscan_body, 0)
            rsmp_v[pl.ds(rcnt, _L)] = junk_vec
            rcol_v[pl.ds(rcnt, _L)] = zero_vec

            dma.wait()
            blk = stage_v.at[r % 2]

            def ext_body(g, cf_in):
                full = cf_in == 128
                pl.when(full)(do_flush)
                cfn = jnp.where(full, 0, cf_in)

                colv = rcol_v[pl.ds(g * _L, _L)]
                smpv = rsmp_v[pl.ds(g * _L, _L)]

                rowv = zero_vec
                paddr = iota * 129
                one = jnp.full((_L,), 1, jnp.int32)
                for kk in range(_E):
                    vals = plsc.load_gather(blk, [rowv, colv])
                    plsc.store_scatter(pitch_v, [paddr], vals)
                    rowv = rowv + one
                    paddr = paddr + one

                for s in range(_L):
                    rvec = jnp.full((_L,), cfn + s, jnp.int32)
                    for q in range(_E // _L):
                        v = pitch_v[pl.ds(129 * s + _L * q, _L)]
                        plsc.store_scatter(f_v, [rvec, col_consts[q]], v)

                plsc.store_scatter(
                    fsmp_v,
                    [jnp.full((_L,), lax.shift_right_logical(cfn, 4),
                              jnp.int32), iota],
                    smpv)
                return cfn + _L

            cf = lax.fori_loop(0, (rcnt + _L - 1) // _L, ext_body, cf)

        @pl.when(cf > 0)
        def _():
            do_flush()

    out_emb = k(info.astype(jnp.int32), tableT)
    return jnp.concatenate([out_emb[:_N, :_E], x], axis=-1)


def kernel(x, info, table):
    return _embed_concat(x, info, table)


# speedup vs baseline: 1.0104x; 1.0104x over previous
import functools

import jax
import jax.numpy as jnp
from jax import lax
from jax.experimental import pallas as pl
from jax.experimental.pallas import tpu as pltpu
from jax.experimental.pallas import tpu_sc as plsc

_N = 4096
_R = 64
_E = 64
_V = 100000
_L = 16
_NT = 782
_TPW = 25
_RT = 4
_NR = 7
_RC = _RT * 128
_JUNK = _N


@jax.jit
def _embed_concat(x, info, table):
    sc = plsc.get_sparse_core_info()
    nc, ns = sc.num_cores, sc.num_subcores
    nw = nc * ns

    tableT = table.T

    mesh = plsc.VectorSubcoreMesh(core_axis_name="c", subcore_axis_name="s")

    @functools.partial(
        pl.kernel,
        mesh=mesh,
        out_type=jax.ShapeDtypeStruct((_N + _L, _E + _R), jnp.float32),
        scratch_types=[
            pltpu.VMEM((_N,), jnp.int32),
            pltpu.VMEM((_N + _L,), jnp.int32),
            pltpu.VMEM((_N + _L,), jnp.int32),
            pltpu.VMEM((_N + _L,), jnp.int32),
            pltpu.VMEM((_N + _L,), jnp.int32),
            pltpu.VMEM((2, _E, _RC), jnp.float32),
            pltpu.VMEM((_L * 129 + _L,), jnp.float32),
            pltpu.VMEM((128, _E + _R), jnp.float32),
            pltpu.VMEM((8, _L), jnp.int32),
            pltpu.SemaphoreType.DMA,
            pltpu.SemaphoreType.DMA,
            pltpu.SemaphoreType.DMA,
        ],
        compiler_params=pltpu.CompilerParams(needs_layout_passes=False),
    )
    def k(idx_hbm, tableT_hbm, out_hbm,
          idx_v, hcol_v, hsmp_v, rcol_v, rsmp_v, stage_v, pitch_v, f_v,
          fsmp_v, sem_a, sem_b, sem_s):
        wid = lax.axis_index("s") * nc + lax.axis_index("c")
        tlo = _TPW * wid
        wtiles = jnp.minimum(_TPW, _NT - tlo)
        clo = tlo * 128
        wlen = wtiles * 128

        iota = lax.iota(jnp.int32, _L)
        junk_vec = jnp.full((_L,), _JUNK, jnp.int32)
        zero_vec = jnp.zeros((_L,), jnp.int32)

        sems = (sem_a, sem_b)

        def round_dma(r):
            c0loc = jnp.maximum(0, jnp.minimum(_RC * r, wlen - _RC))
            gcol = pl.multiple_of(clo + c0loc, 128)
            return c0loc, pltpu.async_copy(
                tableT_hbm.at[:, pl.ds(gcol, _RC)],
                stage_v.at[r % 2],
                sems[r % 2],
            )

        pending = [round_dma(0)]

        pltpu.sync_copy(idx_hbm, idx_v)

        clo_vec = jnp.full((_L,), clo, jnp.int32)
        chi_vec = jnp.full((_L,), clo + wlen, jnp.int32)

        def scan_body(j, off):
            v = idx_v[pl.ds(j * _L, _L)]
            m = (v >= clo_vec) & (v < chi_vec)
            n = jnp.max(plsc.all_reduce_population_count(m))
            plsc.store_compressed(hcol_v.at[pl.ds(off, _L)], v - clo_vec,
                                  mask=m)
            plsc.store_compressed(hsmp_v.at[pl.ds(off, _L)], j * _L + iota,
                                  mask=m)
            return off + n

        cnt = lax.fori_loop(0, _N // _L, scan_body, 0)
        hsmp_v[pl.ds(cnt, _L)] = junk_vec
        hcol_v[pl.ds(cnt, _L)] = zero_vec

        def reset_fsmp():
            for h in range(8):
                plsc.store_scatter(
                    fsmp_v, [jnp.full((_L,), h, jnp.int32), iota], junk_vec)

        reset_fsmp()

        def do_flush():
            copies = []
            for h in range(8):
                copies.append(pltpu.async_copy(
                    f_v.at[pl.ds(_L * h, _L)],
                    out_hbm.at[fsmp_v.at[h]],
                    sem_s,
                ))
            for c in copies:
                c.wait()
            reset_fsmp()

        cf = 0
        for r in range(_NR):
            sbase, dma = pending.pop()
            if r + 1 < _NR:
                pending.append(round_dma(r + 1))

            rlo_vec = jnp.full((_L,), _RC * r, jnp.int32)
            rhi = jnp.minimum(_RC * (r + 1), wlen)
            rhi_vec = jnp.full((_L,), rhi, jnp.int32)
            sbase_vec = jnp.full((_L,), sbase, jnp.int32)

            def rscan_body(j, roff):
                hc = hcol_v[pl.ds(j * _L, _L)]
                hs = hsmp_v[pl.ds(j * _L, _L)]
                m = (hc >= rlo_vec) & (hc < rhi_vec)
                n = jnp.max(plsc.all_reduce_population_count(m))
                plsc.store_compressed(rcol_v.at[pl.ds(roff, _L)],
                                      hc - sbase_vec, mask=m)
                plsc.store_compressed(rsmp_v.at[pl.ds(roff, _L)], hs, mask=m)
                return roff + n

            rcnt = lax.fori_loop(0, (cnt + _L - 1) // _L, rscan_body, 0)
            rsmp_v[pl.ds(rcnt, _L)] = junk_vec
            rcol_v[pl.ds(rcnt, _L)] = zero_vec

            dma.wait()
            blk = stage_v.at[r % 2]

            def ext_body(g, cf_in):
                full = cf_in == 128
                pl.when(full)(do_flush)
                cfn = jnp.where(full, 0, cf_in)

                colv = rcol_v[pl.ds(g * _L, _L)]
                smpv = rsmp_v[pl.ds(g * _L, _L)]

                rowv = zero_vec
                paddr = iota * 129
                one = jnp.full((_L,), 1, jnp.int32)
                for kk in range(_E):
                    vals = plsc.load_gather(blk, [rowv, colv])
                    plsc.store_scatter(pitch_v, [paddr], vals)
                    rowv = rowv + one
                    paddr = paddr + one

                for s in range(_L):
                    for q in range(_E // _L):
                        v = pitch_v[pl.ds(129 * s + _L * q, _L)]
                        f_v[cfn + s, pl.ds(_L * q, _L)] = v

                plsc.store_scatter(
                    fsmp_v,
                    [jnp.full((_L,), lax.shift_right_logical(cfn, 4),
                              jnp.int32), iota],
                    smpv)
                return cfn + _L

            cf = lax.fori_loop(0, (rcnt + _L - 1) // _L, ext_body, cf)

        @pl.when(cf > 0)
        def _():
            do_flush()

    out_emb = k(info.astype(jnp.int32), tableT)
    return jnp.concatenate([out_emb[:_N, :_E], x], axis=-1)


def kernel(x, info, table):
    return _embed_concat(x, info, table)


# speedup vs baseline: 2.2685x; 2.2452x over previous
import functools

import jax
import jax.numpy as jnp
from jax import lax
from jax.experimental import pallas as pl
from jax.experimental.pallas import tpu as pltpu
from jax.experimental.pallas import tpu_sc as plsc

_N = 4096
_R = 64
_E = 64
_L = 16


@jax.jit
def _embed_concat(x, info, table):
    sc = plsc.get_sparse_core_info()
    nc, ns = sc.num_cores, sc.num_subcores
    nw = nc * ns
    b = _N // nw

    table2 = table.reshape(-1, 2 * _E)

    mesh = plsc.VectorSubcoreMesh(core_axis_name="c", subcore_axis_name="s")

    @functools.partial(
        pl.kernel,
        mesh=mesh,
        out_type=jax.ShapeDtypeStruct((_N, _E + _R), jnp.float32),
        scratch_types=[
            pltpu.VMEM((b,), jnp.int32),
            pltpu.VMEM((b,), jnp.int32),
            pltpu.VMEM((b, 2 * _E), jnp.float32),
            pltpu.VMEM((b, _R), jnp.float32),
            pltpu.VMEM((b, _E + _R), jnp.float32),
            pltpu.SemaphoreType.DMA,
        ],
        compiler_params=pltpu.CompilerParams(needs_layout_passes=False),
    )
    def k(x_hbm, idx_hbm, table2_hbm, out_hbm,
          idx_v, jdx_v, emb2_v, x_v, out_v, gsem):
        wid = lax.axis_index("s") * nc + lax.axis_index("c")
        base = wid * b

        pltpu.sync_copy(idx_hbm.at[pl.ds(base, b)], idx_v)

        def halve(t, _):
            v = idx_v[pl.ds(t * _L, _L)]
            jdx_v[pl.ds(t * _L, _L)] = lax.shift_right_logical(v, 1)
            return _

        lax.fori_loop(0, b // _L, halve, 0, unroll=True)

        gather = pltpu.async_copy(table2_hbm.at[jdx_v], emb2_v, gsem)
        pltpu.sync_copy(x_hbm.at[pl.ds(base, b)], x_v)
        gather.wait()

        iota = lax.iota(jnp.int32, _L)
        col_chunks = [c * _L + iota for c in range(_E // _L)]

        def assemble(i, _):
            rowi = jnp.full((_L,), i, jnp.int32)
            vi = plsc.load_gather(idx_v, [rowi])
            off = lax.shift_left(jnp.bitwise_and(vi, 1), 6)
            for t in range(_E // _L):
                val = plsc.load_gather(emb2_v, [rowi, off + col_chunks[t]])
                plsc.store_scatter(out_v, [rowi, col_chunks[t]], val)
                xv = plsc.load_gather(x_v, [rowi, col_chunks[t]])
                plsc.store_scatter(out_v, [rowi, _E + col_chunks[t]], xv)
            return _

        lax.fori_loop(0, b, assemble, 0)

        pltpu.sync_copy(out_v, out_hbm.at[pl.ds(base, b)])

    return k(x, info.astype(jnp.int32), table2)


def kernel(x, info, table):
    return _embed_concat(x, info, table)


# speedup vs baseline: 2.2789x; 1.0046x over previous
import functools

import jax
import jax.numpy as jnp
from jax import lax
from jax.experimental import pallas as pl
from jax.experimental.pallas import tpu as pltpu
from jax.experimental.pallas import tpu_sc as plsc

_N = 4096
_R = 64
_E = 64
_L = 16


@jax.jit
def _embed_concat(x, info, table):
    sc = plsc.get_sparse_core_info()
    nc, ns = sc.num_cores, sc.num_subcores
    nw = nc * ns
    b = _N // nw

    table2 = table.reshape(-1, 2 * _E)

    mesh = plsc.VectorSubcoreMesh(core_axis_name="c", subcore_axis_name="s")

    @functools.partial(
        pl.kernel,
        mesh=mesh,
        out_type=jax.ShapeDtypeStruct((_N, _E + _R), jnp.float32),
        scratch_types=[
            pltpu.VMEM((b,), jnp.int32),
            pltpu.VMEM((b,), jnp.int32),
            pltpu.VMEM((b, 2 * _E), jnp.float32),
            pltpu.VMEM((b, _R), jnp.float32),
            pltpu.VMEM((b, _E + _R), jnp.float32),
            pltpu.SemaphoreType.DMA,
        ],
        compiler_params=pltpu.CompilerParams(needs_layout_passes=False),
    )
    def k(x_hbm, idx_hbm, table2_hbm, out_hbm,
          idx_v, jdx_v, emb2_v, x_v, out_v, gsem):
        wid = lax.axis_index("s") * nc + lax.axis_index("c")
        base = wid * b

        pltpu.sync_copy(idx_hbm.at[pl.ds(base, b)], idx_v)

        def halve(t, _):
            v = idx_v[pl.ds(t * _L, _L)]
            jdx_v[pl.ds(t * _L, _L)] = lax.shift_right_logical(v, 1)
            return _

        lax.fori_loop(0, b // _L, halve, 0, unroll=True)

        gather = pltpu.async_copy(table2_hbm.at[jdx_v], emb2_v, gsem)
        pltpu.sync_copy(x_hbm.at[pl.ds(base, b)], x_v)
        gather.wait()

        iota = lax.iota(jnp.int32, _L)
        col_chunks = [c * _L + iota for c in range(_E // _L)]

        def assemble(i, _):
            rowi = jnp.full((_L,), i, jnp.int32)
            vi = plsc.load_gather(idx_v, [rowi])
            off = lax.shift_left(jnp.bitwise_and(vi, 1), 6)
            for t in range(_E // _L):
                val = plsc.load_gather(emb2_v, [rowi, off + col_chunks[t]])
                out_v[i, pl.ds(t * _L, _L)] = val
                out_v[i, pl.ds(_E + t * _L, _L)] = x_v[i, pl.ds(t * _L, _L)]
            return _

        lax.fori_loop(0, b, assemble, 0)

        pltpu.sync_copy(out_v, out_hbm.at[pl.ds(base, b)])

    return k(x, info.astype(jnp.int32), table2)


def kernel(x, info, table):
    return _embed_concat(x, info, table)
